# Initial kernel scaffold; baseline (speedup 1.0000x reference)
import jax
import jax.numpy as jnp
from jax.experimental import pallas as pl


def kernel(W):
    raise NotImplementedError("write your pallas kernel here")


# speedup vs baseline: 2.9395x; 2.9395x over previous
import functools

import jax
import jax.numpy as jnp
from jax import lax
from jax.experimental import pallas as pl
from jax.experimental.pallas import tpu as pltpu

_BI = 256


def _pass1_body(w_ref, abf_ref, u_ref, vp_ref):
    i = pl.program_id(1)
    a = jnp.abs(w_ref[...])
    abf_ref[...] = a.astype(jnp.bfloat16)
    s = jnp.sum(a, axis=1, keepdims=True)
    u_blk = 1.0 / s
    u_ref[...] = u_blk

    @pl.when(i == 0)
    def _():
        vp_ref[...] = jnp.zeros_like(vp_ref)

    vp_ref[...] += jnp.sum(a * u_blk, axis=0, keepdims=True)[None]


def _iter_body(abf_ref, vpin_ref, u_ref, vp_ref, v_scr):
    i = pl.program_id(1)

    @pl.when(i == 0)
    def _():
        v_scr[...] = 1.0 / (vpin_ref[0] + vpin_ref[1])
        vp_ref[...] = jnp.zeros_like(vp_ref)

    a = abf_ref[...].astype(jnp.float32)
    s = jnp.sum(a * v_scr[...], axis=1, keepdims=True)
    u_blk = 1.0 / s
    u_ref[...] = u_blk
    vp_ref[...] += jnp.sum(a * u_blk, axis=0, keepdims=True)[None]


def _final_body(abf_ref, u_ref, vpin_ref, out_ref, v_scr):
    i = pl.program_id(1)

    @pl.when(i == 0)
    def _():
        v_scr[...] = 1.0 / (vpin_ref[0] + vpin_ref[1])

    out_ref[...] = abf_ref[...].astype(jnp.float32) * u_ref[...] * v_scr[...]


@functools.partial(jax.jit, static_argnums=())
def kernel(W):
    n = W.shape[0]
    bi = min(_BI, n // 2)
    ni = n // (2 * bi)
    grid = (2, ni)
    params = pltpu.CompilerParams(
        dimension_semantics=("parallel", "arbitrary"))

    blk_mat = lambda dt: pl.BlockSpec((bi, n), lambda c, i: (c * ni + i, 0))
    blk_u = pl.BlockSpec((bi, 1), lambda c, i: (c * ni + i, 0))
    blk_vp_out = pl.BlockSpec((1, 1, n), lambda c, i: (c, 0, 0))
    blk_vp_in = pl.BlockSpec((2, 1, n), lambda c, i: (0, 0, 0))

    abf, u, vp = pl.pallas_call(
        _pass1_body,
        grid=grid,
        in_specs=[blk_mat(jnp.float32)],
        out_specs=[blk_mat(jnp.bfloat16), blk_u, blk_vp_out],
        out_shape=[
            jax.ShapeDtypeStruct((n, n), jnp.bfloat16),
            jax.ShapeDtypeStruct((n, 1), jnp.float32),
            jax.ShapeDtypeStruct((2, 1, n), jnp.float32),
        ],
        compiler_params=params,
    )(W)

    iter_call = pl.pallas_call(
        _iter_body,
        grid=grid,
        in_specs=[blk_mat(jnp.bfloat16), blk_vp_in],
        out_specs=[blk_u, blk_vp_out],
        out_shape=[
            jax.ShapeDtypeStruct((n, 1), jnp.float32),
            jax.ShapeDtypeStruct((2, 1, n), jnp.float32),
        ],
        scratch_shapes=[pltpu.VMEM((1, n), jnp.float32)],
        compiler_params=params,
    )

    u, vp = lax.fori_loop(
        0, 9, lambda _, c: iter_call(abf, c[1]), (u, vp))

    out = pl.pallas_call(
        _final_body,
        grid=grid,
        in_specs=[blk_mat(jnp.bfloat16), blk_u, blk_vp_in],
        out_specs=blk_mat(jnp.float32),
        out_shape=jax.ShapeDtypeStruct((n, n), jnp.float32),
        scratch_shapes=[pltpu.VMEM((1, n), jnp.float32)],
        compiler_params=params,
    )(abf, u, vp)
    return out
